# baseline (speedup 1.0000x reference)
import functools

import numpy as np

import jax
import jax.numpy as jnp
from jax import lax
from jax.experimental import pallas as pl
from jax.experimental.pallas import tpu as pltpu
from jax.experimental.pallas import tpu_sc as plsc

N_SRC = 10000
N_TGT = 10000
N_EDGE = 320000
D = 128
TF_D = 32

NW = 32
CH = 128
NCHT = N_EDGE // CH
NCH = 80
NCHR = NCHT - 31 * NCH
PADR = 32 * NCH - NCHT
DEG_N = 10240
DEG_RPT = DEG_N // 16
ACC_N = 10112
RPT = ACC_N // 16
PH = 2
PCH = NCH // PH


def _sc_mesh():
    return plsc.VectorSubcoreMesh(core_axis_name="c", subcore_axis_name="s")


def _deg_count(iei):
    @functools.partial(
        pl.kernel,
        out_type=jax.ShapeDtypeStruct((2, DEG_N), jnp.float32),
        mesh=_sc_mesh(),
        scratch_types=[
            pltpu.VMEM((NCH, CH), jnp.int32),
            pltpu.VMEM((CH,), jnp.float32),
            pltpu.VMEM((DEG_RPT,), jnp.float32),
            pltpu.VMEM_SHARED((DEG_N,), jnp.float32),
            pltpu.SemaphoreType.DMA,
            pltpu.SemaphoreType.DMA,
        ],
    )
    def k(iei_hbm, out_hbm, idx_v, ones_v, zst_v, deg_sh, semi, sem):
        c = lax.axis_index("c")
        s = lax.axis_index("s")
        w = c * 16 + s
        nch_w = jnp.where(w < NW - 1, NCH, NCHR)
        for k in range(NCH):
            @pl.when(w * NCH + k < NCHT)
            def _():
                pltpu.async_copy(
                    iei_hbm.at[0, pl.ds((w * NCH + k) * CH, CH)],
                    idx_v.at[k], semi)
        for j in range(CH // 16):
            ones_v[pl.ds(j * 16, 16)] = jnp.ones((16,), jnp.float32)
        for j in range(DEG_RPT // 16):
            zst_v[pl.ds(j * 16, 16)] = jnp.zeros((16,), jnp.float32)
        pltpu.sync_copy(zst_v, deg_sh.at[pl.ds(s * DEG_RPT, DEG_RPT)])
        for k in range(NCH):
            @pl.when(w * NCH + k < NCHT)
            def _():
                pltpu.make_async_copy(
                    iei_hbm.at[0, pl.ds((w * NCH + k) * CH, CH)],
                    idx_v.at[k], semi).wait()
        plsc.subcore_barrier()

        def body(i, _):
            cps = [
                pltpu.async_copy(ones_v, deg_sh.at[idx_v.at[i * 4 + g]], sem,
                                 add=True)
                for g in range(4)
            ]
            for cp in cps:
                cp.wait()
            return 0

        lax.fori_loop(0, nch_w // 4, body, 0)
        plsc.subcore_barrier()

        @pl.when(s == 0)
        def _():
            pltpu.sync_copy(deg_sh, out_hbm.at[c])

    return k(iei)


def _scale_matmul(degp, x, w):
    def body(d_ref, x_ref, w_ref, o_ref):
        dis = lax.rsqrt(d_ref[0] + d_ref[1] + 1.0)
        o_ref[...] = jnp.dot(x_ref[...] * dis, w_ref[...],
                             preferred_element_type=jnp.float32)

    return pl.pallas_call(
        body,
        out_shape=jax.ShapeDtypeStruct((N_SRC, D), jnp.float32),
    )(degp, x, w)


def _aggregate(hs, iei, col2d):
    @functools.partial(
        pl.kernel,
        out_type=jax.ShapeDtypeStruct((2, ACC_N, D), jnp.float32),
        mesh=_sc_mesh(),
        scratch_types=[
            pltpu.VMEM((PCH * CH,), jnp.int32),
            pltpu.VMEM((PCH, CH), jnp.int32),
            pltpu.VMEM((CH, D), jnp.float32),
            pltpu.VMEM((CH, D), jnp.float32),
            pltpu.VMEM_SHARED((ACC_N, D), jnp.float32),
            pltpu.SemaphoreType.DMA,
            pltpu.SemaphoreType.DMA,
            pltpu.SemaphoreType.DMA,
        ],
    )
    def k(hs_hbm, iei_hbm, col_hbm, out_hbm, idx_r, idx_c, rows_a, rows_b,
          acc_sh, semi, sem_a, sem_b):
        c = lax.axis_index("c")
        s = lax.axis_index("s")
        w = c * 16 + s

        def zidx(i, _):
            idx_r[pl.ds(i * 16, 16)] = jnp.zeros((16,), jnp.int32)
            return 0

        lax.fori_loop(0, PCH * CH // 16, zidx, 0)

        def zrow(i, _):
            for j in range(D // 16):
                rows_a[i, pl.ds(j * 16, 16)] = jnp.zeros((16,), jnp.float32)
            return 0

        lax.fori_loop(0, CH, zrow, 0)
        for q in range(RPT // CH):
            pltpu.sync_copy(rows_a, acc_sh.at[pl.ds(s * RPT + q * CH, CH)])
        rem = RPT - (RPT // CH) * CH
        pltpu.sync_copy(rows_a.at[pl.ds(0, rem)],
                        acc_sh.at[pl.ds(s * RPT + RPT - rem, rem)])
        plsc.subcore_barrier()

        for p in range(PH):
            pairs = PCH // 2
            fbase = (w * NCH + p * PCH) * CH

            @pl.when(w < NW - 1)
            def _():
                pltpu.sync_copy(iei_hbm.at[0, pl.ds(fbase, PCH * CH)], idx_r)

            if p == 0:
                @pl.when(w == NW - 1)
                def _():
                    pltpu.sync_copy(iei_hbm.at[0, pl.ds(fbase, NCHR * CH)],
                                    idx_r.at[pl.ds(0, NCHR * CH)])
            pltpu.sync_copy(col_hbm.at[pl.ds(w * NCH + p * PCH, PCH)], idx_c)
            pltpu.async_copy(hs_hbm.at[idx_r.at[pl.ds(0, CH)]], rows_a, sem_a)

            def body(j, _):
                pltpu.async_copy(
                    hs_hbm.at[idx_r.at[pl.ds((2 * j + 1) * CH, CH)]], rows_b,
                    sem_b)
                pltpu.make_async_copy(
                    hs_hbm.at[idx_r.at[pl.ds(2 * j * CH, CH)]], rows_a,
                    sem_a).wait()
                pltpu.sync_copy(rows_a, acc_sh.at[idx_c.at[2 * j]], add=True)

                @pl.when(j < pairs - 1)
                def _():
                    pltpu.async_copy(
                        hs_hbm.at[idx_r.at[pl.ds((2 * j + 2) * CH, CH)]],
                        rows_a, sem_a)

                pltpu.make_async_copy(
                    hs_hbm.at[idx_r.at[pl.ds((2 * j + 1) * CH, CH)]], rows_b,
                    sem_b).wait()
                pltpu.sync_copy(rows_b, acc_sh.at[idx_c.at[2 * j + 1]],
                                add=True)
                return 0

            lax.fori_loop(0, pairs, body, 0)

        plsc.subcore_barrier()
        pltpu.sync_copy(acc_sh.at[pl.ds(s * RPT, RPT)],
                        out_hbm.at[c, pl.ds(s * RPT, RPT)])

    return k(hs, iei, col2d)


def _finalize(acc, b, tf):
    def body(a_ref, b_ref, t_ref, o_ref):
        y = a_ref[0, :N_TGT, :] + a_ref[1, :N_TGT, :] + b_ref[...]
        o_ref[:, :D] = jnp.maximum(y, 0.0)
        o_ref[:, D:] = jnp.abs(t_ref[...])

    return pl.pallas_call(
        body,
        out_shape=jax.ShapeDtypeStruct((N_TGT, D + TF_D), jnp.float32),
    )(acc, b, tf)


def kernel(x, inter_edge_index, W, b, target_feat):
    pad = np.arange(PADR * CH, dtype=np.int32).reshape(PADR, CH)
    col2d = jnp.concatenate(
        [inter_edge_index[1].reshape(NCHT, CH),
         jnp.asarray(N_TGT + pad % (ACC_N - N_TGT))])
    degp = _deg_count(inter_edge_index)
    hs = _scale_matmul(degp[:, :N_SRC, None], x, W)
    acc = _aggregate(hs, inter_edge_index, col2d)
    return _finalize(acc, b, target_feat)


# speedup vs baseline: 2.2496x; 2.2496x over previous
import functools

import jax
import jax.numpy as jnp
from jax import lax
from jax.experimental import pallas as pl
from jax.experimental.pallas import tpu as pltpu
from jax.experimental.pallas import tpu_sc as plsc

N_SRC = 10000
N_TGT = 10000
N_EDGE = 320000
D = 128
TF_D = 32

NW = 32
CH = 128
NCHT = N_EDGE // CH
NCH = 80
NCHR = NCHT - 31 * NCH
PADR = 32 * NCH - NCHT
DEG_N = 10240
DEG_RPT = DEG_N // 16
ACC_N = 10112
RPT = ACC_N // 16
PH = 2
PCH = NCH // PH


def _sc_mesh():
    return plsc.VectorSubcoreMesh(core_axis_name="c", subcore_axis_name="s")


def _deg_count(row2d):
    @functools.partial(
        pl.kernel,
        out_type=jax.ShapeDtypeStruct((2, DEG_N), jnp.float32),
        mesh=_sc_mesh(),
        scratch_types=[
            pltpu.VMEM((NCH, CH), jnp.int32),
            pltpu.VMEM((CH,), jnp.float32),
            pltpu.VMEM((DEG_RPT,), jnp.float32),
            pltpu.VMEM_SHARED((DEG_N,), jnp.float32),
            pltpu.SemaphoreType.DMA,
            pltpu.SemaphoreType.DMA,
        ],
    )
    def k(row_hbm, out_hbm, idx_v, ones_v, zst_v, deg_sh, semi, sem):
        c = lax.axis_index("c")
        s = lax.axis_index("s")
        w = c * 16 + s
        nch_w = jnp.where(w < NW - 1, NCH, NCHR)
        for j in range(CH // 16):
            ones_v[pl.ds(j * 16, 16)] = jnp.ones((16,), jnp.float32)
        for j in range(DEG_RPT // 16):
            zst_v[pl.ds(j * 16, 16)] = jnp.zeros((16,), jnp.float32)
        idx_cp = pltpu.async_copy(row_hbm.at[pl.ds(w * NCH, NCH)], idx_v, semi)
        pltpu.sync_copy(zst_v, deg_sh.at[pl.ds(s * DEG_RPT, DEG_RPT)])
        idx_cp.wait()
        plsc.subcore_barrier()

        def body(i, _):
            cps = [
                pltpu.async_copy(ones_v, deg_sh.at[idx_v.at[i * 4 + g]], sem,
                                 add=True)
                for g in range(4)
            ]
            for cp in cps:
                cp.wait()
            return 0

        lax.fori_loop(0, nch_w // 4, body, 0)
        plsc.subcore_barrier()

        @pl.when(s == 0)
        def _():
            pltpu.sync_copy(deg_sh, out_hbm.at[c])

    return k(row2d)


def _scale_matmul(degp, x, w):
    def body(d_ref, x_ref, w_ref, o_ref):
        dis = lax.rsqrt(d_ref[0] + d_ref[1] + 1.0)
        o_ref[...] = jnp.dot(x_ref[...] * dis, w_ref[...],
                             preferred_element_type=jnp.float32)

    return pl.pallas_call(
        body,
        out_shape=jax.ShapeDtypeStruct((N_SRC, D), jnp.float32),
    )(degp, x, w)


def _aggregate(hs, row2d, col2d):
    @functools.partial(
        pl.kernel,
        out_type=jax.ShapeDtypeStruct((2, ACC_N, D), jnp.float32),
        mesh=_sc_mesh(),
        scratch_types=[
            pltpu.VMEM((PCH, CH), jnp.int32),
            pltpu.VMEM((PCH, CH), jnp.int32),
            pltpu.VMEM((CH, D), jnp.float32),
            pltpu.VMEM((CH, D), jnp.float32),
            pltpu.VMEM_SHARED((ACC_N, D), jnp.float32),
            pltpu.SemaphoreType.DMA,
            pltpu.SemaphoreType.DMA,
            pltpu.SemaphoreType.DMA,
        ],
    )
    def k(hs_hbm, row_hbm, col_hbm, out_hbm, idx_r, idx_c, rows_a, rows_b,
          acc_sh, semi, sem_a, sem_b):
        c = lax.axis_index("c")
        s = lax.axis_index("s")
        w = c * 16 + s

        def zrow(i, _):
            for j in range(D // 16):
                rows_a[i, pl.ds(j * 16, 16)] = jnp.zeros((16,), jnp.float32)
            return 0

        lax.fori_loop(0, CH, zrow, 0)
        for q in range(RPT // CH):
            pltpu.sync_copy(rows_a, acc_sh.at[pl.ds(s * RPT + q * CH, CH)])
        rem = RPT - (RPT // CH) * CH
        pltpu.sync_copy(rows_a.at[pl.ds(0, rem)],
                        acc_sh.at[pl.ds(s * RPT + RPT - rem, rem)])
        plsc.subcore_barrier()

        for p in range(PH):
            pairs = PCH // 2
            pltpu.sync_copy(row_hbm.at[pl.ds(w * NCH + p * PCH, PCH)], idx_r)
            pltpu.sync_copy(col_hbm.at[pl.ds(w * NCH + p * PCH, PCH)], idx_c)
            pltpu.async_copy(hs_hbm.at[idx_r.at[0]], rows_a, sem_a)

            def body(j, _):
                pltpu.async_copy(hs_hbm.at[idx_r.at[2 * j + 1]], rows_b, sem_b)
                pltpu.make_async_copy(hs_hbm.at[idx_r.at[2 * j]], rows_a,
                                      sem_a).wait()
                pltpu.sync_copy(rows_a, acc_sh.at[idx_c.at[2 * j]], add=True)

                @pl.when(j < pairs - 1)
                def _():
                    pltpu.async_copy(hs_hbm.at[idx_r.at[2 * j + 2]], rows_a,
                                     sem_a)

                pltpu.make_async_copy(hs_hbm.at[idx_r.at[2 * j + 1]], rows_b,
                                      sem_b).wait()
                pltpu.sync_copy(rows_b, acc_sh.at[idx_c.at[2 * j + 1]],
                                add=True)
                return 0

            lax.fori_loop(0, pairs, body, 0)

        plsc.subcore_barrier()
        pltpu.sync_copy(acc_sh.at[pl.ds(s * RPT, RPT)],
                        out_hbm.at[c, pl.ds(s * RPT, RPT)])

    return k(hs, row2d, col2d)


def _finalize(acc, b, tf):
    def body(a_ref, b_ref, t_ref, o_ref):
        y = a_ref[0, :N_TGT, :] + a_ref[1, :N_TGT, :] + b_ref[...]
        o_ref[:, :D] = jnp.maximum(y, 0.0)
        o_ref[:, D:] = jnp.abs(t_ref[...])

    return pl.pallas_call(
        body,
        out_shape=jax.ShapeDtypeStruct((N_TGT, D + TF_D), jnp.float32),
    )(acc, b, tf)


def kernel(x, inter_edge_index, W, b, target_feat):
    pad = jnp.arange(PADR * CH, dtype=inter_edge_index.dtype)
    row2d = jnp.concatenate([inter_edge_index[0], pad % N_SRC]).reshape(-1, CH)
    col2d = jnp.concatenate(
        [inter_edge_index[1], N_TGT + pad % (ACC_N - N_TGT)]).reshape(-1, CH)
    degp = _deg_count(row2d)
    hs = _scale_matmul(degp[:, :N_SRC, None], x, W)
    acc = _aggregate(hs, row2d, col2d)
    return _finalize(acc, b, target_feat)
